# baseline (speedup 1.0000x reference)
import functools

import jax
import jax.numpy as jnp
from jax import lax
from jax.experimental import pallas as pl
from jax.experimental.pallas import tpu as pltpu
from jax.experimental.pallas import tpu_sc as plsc

F32 = jnp.float32
_NW = 32
_CH = 128
_RNG = 3200


def _mesh():
    return plsc.VectorSubcoreMesh(core_axis_name="c", subcore_axis_name="s")


def _wid():
    return lax.axis_index("s") * 2 + lax.axis_index("c")


def _sc_gather_winner(sess_tab, W_emb, user, input_item, pos_items, neg_items):
    B = user.shape[0]
    bw_u = B // _NW
    bw_i = B // _NW
    n_chunks = (bw_u + 3 * bw_i) // _CH
    n_vec = B // 16
    seg = n_vec // n_chunks

    @functools.partial(
        pl.kernel,
        out_type=(
            jax.ShapeDtypeStruct((B, 2, 128), F32),
            jax.ShapeDtypeStruct((3 * B, 128), F32),
            jax.ShapeDtypeStruct((_NW * _RNG,), jnp.int32),
        ),
        mesh=_mesh(),
        compiler_params=pltpu.CompilerParams(needs_layout_passes=False),
        scratch_types=[
            pltpu.VMEM((bw_u,), jnp.int32),
            pltpu.VMEM((3 * bw_i,), jnp.int32),
            pltpu.VMEM((B,), jnp.int32),
            pltpu.VMEM((_RNG,), jnp.int32),
            pltpu.VMEM((2, _CH, 2, 128), F32),
            pltpu.VMEM((2, _CH, 128), F32),
            pltpu.SemaphoreType.DMA,
            pltpu.SemaphoreType.DMA,
            pltpu.SemaphoreType.DMA,
            pltpu.SemaphoreType.DMA,
        ],
    )
    def k(sess_hbm, emb_hbm, user_hbm, in_hbm, pos_hbm, neg_hbm,
          sess_out, xpn_out, win_out,
          uidx, iidx, uall, wloc, sbuf, ebuf, g0, g1, w0, w1):
        wid = _wid()
        ubase = pl.multiple_of(wid * bw_u, _CH)
        ibase = pl.multiple_of(wid * bw_i, _CH)
        lo = pl.multiple_of(wid * _RNG, _RNG)
        pltpu.sync_copy(user_hbm.at[pl.ds(ubase, bw_u)], uidx)
        for r, hbm in enumerate((in_hbm, pos_hbm, neg_hbm)):
            pltpu.sync_copy(hbm.at[pl.ds(ibase, bw_i)],
                            iidx.at[pl.ds(r * bw_i, bw_i)])
        pltpu.sync_copy(user_hbm, uall)
        gsem = (g0, g1)
        wsem = (w0, w1)

        lane = lax.iota(jnp.int32, 16)
        perm = lax.bitwise_and(lane + 1, 15)

        def win_body(j, carry):
            u = uall[pl.ds(pl.multiple_of(j * 16, 16), 16)]
            comb = u * 16384 + (j * 16 + lane)
            sk, _ = plsc.sort_key_val(comb, comb)
            us = lax.shift_right_arithmetic(sk, 14)
            ps = lax.bitwise_and(sk, 16383)
            dnums = lax.GatherDimensionNumbers(
                offset_dims=(), collapsed_slice_dims=(0,), start_index_map=(0,))
            un = lax.gather(us, perm[:, None], dnums, slice_sizes=(1,),
                            mode=lax.GatherScatterMode.PROMISE_IN_BOUNDS)
            is_last = jnp.logical_or(lane == 15, us != un)
            inr = jnp.logical_and(us >= lo, us < lo + _RNG)
            plsc.store_scatter(wloc, [us - lo], ps,
                               mask=jnp.logical_and(is_last, inr))
            return carry

        seg_no = [0]

        def winner_segment():
            t = seg_no[0]
            if t < n_chunks:
                @pl.loop(t * seg, (t + 1) * seg, unroll=2)
                def _(j):
                    win_body(j, 0)
            seg_no[0] += 1

        def pipeline(n, tab, idx, buf, out, obase_fn):
            gh = [None] * n
            wh = [None] * n
            gh[0] = pltpu.async_copy(tab.at[idx.at[pl.ds(0, _CH)]], buf.at[0], gsem[0])
            for j in range(n):
                b = j & 1
                gh[j].wait()
                dst = out.at[pl.ds(pl.multiple_of(obase_fn(j), _CH), _CH)]
                wh[j] = pltpu.async_copy(buf.at[b], dst, wsem[b])
                if j + 1 < n:
                    if j >= 1:
                        wh[j - 1].wait()
                    gh[j + 1] = pltpu.async_copy(
                        tab.at[idx.at[pl.ds((j + 1) * _CH, _CH)]],
                        buf.at[(j + 1) & 1], gsem[(j + 1) & 1])
                winner_segment()
            if n >= 2:
                wh[n - 2].wait()
            wh[n - 1].wait()

        nu = bw_u // _CH
        ni = bw_i // _CH
        pipeline(nu, sess_hbm, uidx, sbuf, sess_out,
                 lambda j: ubase + j * _CH)
        pipeline(3 * ni, emb_hbm, iidx, ebuf, xpn_out,
                 lambda j: (j // ni) * B + ibase + (j % ni) * _CH)
        pltpu.sync_copy(wloc, win_out.at[pl.ds(lo, _RNG)])

    return k(sess_tab, W_emb, user, input_item, pos_items, neg_items)


def _tc_gru(sess, xpn, w_ih, w_hh, b_ih, b_hh):
    B = sess.shape[0]
    BB = 2048
    nb = B // BB
    dn = (((1,), (1,)), ((), ()))

    def body(sess_ref, x_ref, p_ref, n_ref, wih_ref, whh_ref, bih_ref, bhh_ref,
             out_ref, sc_ref):
        h_in = x_ref[...]
        hs = []
        for l in range(2):
            h_prev = sess_ref[:, l, :]
            gi = lax.dot_general(h_in, wih_ref[l], dn,
                                 preferred_element_type=F32) + bih_ref[l][None, :]
            gh = lax.dot_general(h_prev, whh_ref[l], dn,
                                 preferred_element_type=F32) + bhh_ref[l][None, :]
            r = 1.0 / (1.0 + jnp.exp(-(gi[:, 0:128] + gh[:, 0:128])))
            z = 1.0 / (1.0 + jnp.exp(-(gi[:, 128:256] + gh[:, 128:256])))
            pre_n = gi[:, 256:384] + r * gh[:, 256:384]
            n = 2.0 / (1.0 + jnp.exp(-2.0 * pre_n)) - 1.0
            h_in = n + z * (h_prev - n)
            hs.append(h_in)
        out_ref[:, 0, :] = hs[0]
        out_ref[:, 1, :] = hs[1]
        sc_ref[...] = jnp.sum(h_in * (p_ref[...] - n_ref[...]),
                              axis=-1, keepdims=True)

    return pl.pallas_call(
        body,
        grid=(nb,),
        in_specs=[
            pl.BlockSpec((BB, 2, 128), lambda i: (i, 0, 0)),
            pl.BlockSpec((BB, 128), lambda i: (i, 0)),
            pl.BlockSpec((BB, 128), lambda i, _nb=nb: (i + _nb, 0)),
            pl.BlockSpec((BB, 128), lambda i, _nb=nb: (i + 2 * _nb, 0)),
            pl.BlockSpec((2, 384, 128), lambda i: (0, 0, 0)),
            pl.BlockSpec((2, 384, 128), lambda i: (0, 0, 0)),
            pl.BlockSpec((2, 384), lambda i: (0, 0)),
            pl.BlockSpec((2, 384), lambda i: (0, 0)),
        ],
        out_specs=[
            pl.BlockSpec((BB, 2, 128), lambda i: (i, 0, 0)),
            pl.BlockSpec((BB, 1), lambda i: (i, 0)),
        ],
        out_shape=[
            jax.ShapeDtypeStruct((B, 2, 128), F32),
            jax.ShapeDtypeStruct((B, 1), F32),
        ],
    )(sess, xpn, xpn, xpn, w_ih, w_hh, b_ih, b_hh)


def _tc_copy(tab):
    R = tab.shape[0]
    BR = 5000

    def body(in_ref, out_ref):
        out_ref[...] = in_ref[...]

    return pl.pallas_call(
        body,
        grid=(R // BR,),
        in_specs=[pl.BlockSpec((BR, 2, 128), lambda i: (i, 0, 0))],
        out_specs=pl.BlockSpec((BR, 2, 128), lambda i: (i, 0, 0)),
        out_shape=jax.ShapeDtypeStruct((R, 2, 128), F32),
    )(tab)


def _sc_scatter(upd, user, win, new_rows):
    B = user.shape[0]
    bw = B // _NW

    n = bw // _CH

    @functools.partial(
        pl.kernel,
        mesh=_mesh(),
        scratch_types=[
            pltpu.VMEM((n, _CH), jnp.int32),
            pltpu.VMEM((n, _CH), jnp.int32),
            pltpu.VMEM((2, _CH, 2, 128), F32),
            pltpu.SemaphoreType.DMA,
            pltpu.SemaphoreType.DMA,
        ],
    )
    def k(out_hbm, user_hbm, win_hbm, rows_hbm, uidx, sel, row, sem0, sem1):
        wid = _wid()
        sems = (sem0, sem1)

        for j in range(n):
            base = pl.multiple_of(wid * bw + j * _CH, _CH)
            pltpu.sync_copy(user_hbm.at[pl.ds(base, _CH)], uidx.at[j])
        sh = [pltpu.async_copy(win_hbm.at[uidx.at[j]], sel.at[j], sem0)
              for j in range(n)]
        for h in sh:
            h.wait()

        gh = [None] * n
        wh = [None] * n
        gh[0] = pltpu.async_copy(rows_hbm.at[sel.at[0]], row.at[0], sems[0])
        for j in range(n):
            b = j & 1
            gh[j].wait()
            wh[j] = pltpu.async_copy(row.at[b], out_hbm.at[uidx.at[j]], sems[b])
            if j + 1 < n:
                if j >= 1:
                    wh[j - 1].wait()
                gh[j + 1] = pltpu.async_copy(
                    rows_hbm.at[sel.at[j + 1]], row.at[(j + 1) & 1],
                    sems[(j + 1) & 1])
        if n >= 2:
            wh[n - 2].wait()
        wh[n - 1].wait()

    ref = jax.new_ref(upd)
    k(ref, user, win, new_rows)
    return ref[...]


def kernel(user, input_item, pos_items, neg_items, user_sessions, W_emb,
           w_ih, w_hh, b_ih, b_hh):
    user = user.astype(jnp.int32)
    sess, xpn, win = _sc_gather_winner(
        user_sessions, W_emb, user, input_item.astype(jnp.int32),
        pos_items.astype(jnp.int32), neg_items.astype(jnp.int32))
    new_rows, scores = _tc_gru(sess, xpn, w_ih, w_hh, b_ih, b_hh)
    upd = _tc_copy(user_sessions)
    return scores, _sc_scatter(upd, user, win, new_rows)


# speedup vs baseline: 1.0278x; 1.0278x over previous
import functools

import jax
import jax.numpy as jnp
from jax import lax
from jax.experimental import pallas as pl
from jax.experimental.pallas import tpu as pltpu
from jax.experimental.pallas import tpu_sc as plsc

F32 = jnp.float32
_NW = 32
_CH = 128
_RNG = 3200


def _mesh():
    return plsc.VectorSubcoreMesh(core_axis_name="c", subcore_axis_name="s")


def _wid():
    return lax.axis_index("s") * 2 + lax.axis_index("c")


def _sc_gather_winner(sess_tab, W_emb, user, input_item, pos_items, neg_items):
    B = user.shape[0]
    bw_u = B // _NW
    bw_i = B // _NW
    n_chunks = (bw_u + 3 * bw_i) // _CH
    n_vec = B // 16
    seg = n_vec // n_chunks

    @functools.partial(
        pl.kernel,
        out_type=(
            jax.ShapeDtypeStruct((B, 2, 128), F32),
            jax.ShapeDtypeStruct((3 * B, 128), F32),
            jax.ShapeDtypeStruct((_NW * _RNG,), jnp.int32),
        ),
        mesh=_mesh(),
        compiler_params=pltpu.CompilerParams(needs_layout_passes=False),
        scratch_types=[
            pltpu.VMEM((bw_u,), jnp.int32),
            pltpu.VMEM((3 * bw_i,), jnp.int32),
            pltpu.VMEM((B,), jnp.int32),
            pltpu.VMEM((_RNG,), jnp.int32),
            pltpu.VMEM((2, _CH, 2, 128), F32),
            pltpu.VMEM((2, _CH, 128), F32),
            pltpu.SemaphoreType.DMA,
            pltpu.SemaphoreType.DMA,
            pltpu.SemaphoreType.DMA,
            pltpu.SemaphoreType.DMA,
        ],
    )
    def k(sess_hbm, emb_hbm, user_hbm, in_hbm, pos_hbm, neg_hbm,
          sess_out, xpn_out, win_out,
          uidx, iidx, uall, wloc, sbuf, ebuf, g0, g1, w0, w1):
        wid = _wid()
        ubase = pl.multiple_of(wid * bw_u, _CH)
        ibase = pl.multiple_of(wid * bw_i, _CH)
        lo = pl.multiple_of(wid * _RNG, _RNG)
        pltpu.sync_copy(user_hbm.at[pl.ds(ubase, bw_u)], uidx)
        for r, hbm in enumerate((in_hbm, pos_hbm, neg_hbm)):
            pltpu.sync_copy(hbm.at[pl.ds(ibase, bw_i)],
                            iidx.at[pl.ds(r * bw_i, bw_i)])
        pltpu.sync_copy(user_hbm, uall)
        gsem = (g0, g1)
        wsem = (w0, w1)

        lane = lax.iota(jnp.int32, 16)
        perm = lax.bitwise_and(lane + 1, 15)

        def win_body(j, carry):
            u = uall[pl.ds(pl.multiple_of(j * 16, 16), 16)]
            ps = j * 16 + lane
            inr = jnp.logical_and(u >= lo, u < lo + _RNG)
            plsc.store_scatter(wloc, [u - lo], ps, mask=inr)
            return carry

        seg_no = [0]

        def winner_segment():
            t = seg_no[0]
            if t < n_chunks:
                @pl.loop(t * seg, (t + 1) * seg, unroll=2)
                def _(j):
                    win_body(j, 0)
            seg_no[0] += 1

        def pipeline(n, tab, idx, buf, out, obase_fn):
            gh = [None] * n
            wh = [None] * n
            gh[0] = pltpu.async_copy(tab.at[idx.at[pl.ds(0, _CH)]], buf.at[0], gsem[0])
            for j in range(n):
                b = j & 1
                gh[j].wait()
                dst = out.at[pl.ds(pl.multiple_of(obase_fn(j), _CH), _CH)]
                wh[j] = pltpu.async_copy(buf.at[b], dst, wsem[b])
                if j + 1 < n:
                    if j >= 1:
                        wh[j - 1].wait()
                    gh[j + 1] = pltpu.async_copy(
                        tab.at[idx.at[pl.ds((j + 1) * _CH, _CH)]],
                        buf.at[(j + 1) & 1], gsem[(j + 1) & 1])
                winner_segment()
            if n >= 2:
                wh[n - 2].wait()
            wh[n - 1].wait()

        nu = bw_u // _CH
        ni = bw_i // _CH
        pipeline(nu, sess_hbm, uidx, sbuf, sess_out,
                 lambda j: ubase + j * _CH)
        pipeline(3 * ni, emb_hbm, iidx, ebuf, xpn_out,
                 lambda j: (j // ni) * B + ibase + (j % ni) * _CH)
        pltpu.sync_copy(wloc, win_out.at[pl.ds(lo, _RNG)])

    return k(sess_tab, W_emb, user, input_item, pos_items, neg_items)


def _tc_gru(sess, xpn, w_ih, w_hh, b_ih, b_hh):
    B = sess.shape[0]
    BB = 2048
    nb = B // BB
    dn = (((1,), (1,)), ((), ()))

    def body(sess_ref, x_ref, p_ref, n_ref, wih_ref, whh_ref, bih_ref, bhh_ref,
             out_ref, sc_ref):
        h_in = x_ref[...]
        hs = []
        for l in range(2):
            h_prev = sess_ref[:, l, :]
            gi = lax.dot_general(h_in, wih_ref[l], dn,
                                 preferred_element_type=F32) + bih_ref[l][None, :]
            gh = lax.dot_general(h_prev, whh_ref[l], dn,
                                 preferred_element_type=F32) + bhh_ref[l][None, :]
            r = jax.nn.sigmoid(gi[:, 0:128] + gh[:, 0:128])
            z = jax.nn.sigmoid(gi[:, 128:256] + gh[:, 128:256])
            n = jnp.tanh(gi[:, 256:384] + r * gh[:, 256:384])
            h_in = n + z * (h_prev - n)
            hs.append(h_in)
        out_ref[:, 0, :] = hs[0]
        out_ref[:, 1, :] = hs[1]
        sc_ref[...] = jnp.sum(h_in * (p_ref[...] - n_ref[...]),
                              axis=-1, keepdims=True)

    return pl.pallas_call(
        body,
        grid=(nb,),
        in_specs=[
            pl.BlockSpec((BB, 2, 128), lambda i: (i, 0, 0)),
            pl.BlockSpec((BB, 128), lambda i: (i, 0)),
            pl.BlockSpec((BB, 128), lambda i, _nb=nb: (i + _nb, 0)),
            pl.BlockSpec((BB, 128), lambda i, _nb=nb: (i + 2 * _nb, 0)),
            pl.BlockSpec((2, 384, 128), lambda i: (0, 0, 0)),
            pl.BlockSpec((2, 384, 128), lambda i: (0, 0, 0)),
            pl.BlockSpec((2, 384), lambda i: (0, 0)),
            pl.BlockSpec((2, 384), lambda i: (0, 0)),
        ],
        out_specs=[
            pl.BlockSpec((BB, 2, 128), lambda i: (i, 0, 0)),
            pl.BlockSpec((BB, 1), lambda i: (i, 0)),
        ],
        out_shape=[
            jax.ShapeDtypeStruct((B, 2, 128), F32),
            jax.ShapeDtypeStruct((B, 1), F32),
        ],
    )(sess, xpn, xpn, xpn, w_ih, w_hh, b_ih, b_hh)


def _tc_copy(tab):
    R = tab.shape[0]
    BR = 5000

    def body(in_ref, out_ref):
        out_ref[...] = in_ref[...]

    return pl.pallas_call(
        body,
        grid=(R // BR,),
        in_specs=[pl.BlockSpec((BR, 2, 128), lambda i: (i, 0, 0))],
        out_specs=pl.BlockSpec((BR, 2, 128), lambda i: (i, 0, 0)),
        out_shape=jax.ShapeDtypeStruct((R, 2, 128), F32),
    )(tab)


def _sc_scatter(upd, user, win, new_rows):
    B = user.shape[0]
    bw = B // _NW

    n = bw // _CH

    @functools.partial(
        pl.kernel,
        mesh=_mesh(),
        scratch_types=[
            pltpu.VMEM((n, _CH), jnp.int32),
            pltpu.VMEM((n, _CH), jnp.int32),
            pltpu.VMEM((2, _CH, 2, 128), F32),
            pltpu.SemaphoreType.DMA,
            pltpu.SemaphoreType.DMA,
        ],
    )
    def k(out_hbm, user_hbm, win_hbm, rows_hbm, uidx, sel, row, sem0, sem1):
        wid = _wid()
        sems = (sem0, sem1)

        for j in range(n):
            base = pl.multiple_of(wid * bw + j * _CH, _CH)
            pltpu.sync_copy(user_hbm.at[pl.ds(base, _CH)], uidx.at[j])
        sh = [pltpu.async_copy(win_hbm.at[uidx.at[j]], sel.at[j], sem0)
              for j in range(n)]
        for h in sh:
            h.wait()

        gh = [None] * n
        wh = [None] * n
        gh[0] = pltpu.async_copy(rows_hbm.at[sel.at[0]], row.at[0], sems[0])
        for j in range(n):
            b = j & 1
            gh[j].wait()
            wh[j] = pltpu.async_copy(row.at[b], out_hbm.at[uidx.at[j]], sems[b])
            if j + 1 < n:
                if j >= 1:
                    wh[j - 1].wait()
                gh[j + 1] = pltpu.async_copy(
                    rows_hbm.at[sel.at[j + 1]], row.at[(j + 1) & 1],
                    sems[(j + 1) & 1])
        if n >= 2:
            wh[n - 2].wait()
        wh[n - 1].wait()

    ref = jax.new_ref(upd)
    k(ref, user, win, new_rows)
    return ref[...]


def kernel(user, input_item, pos_items, neg_items, user_sessions, W_emb,
           w_ih, w_hh, b_ih, b_hh):
    user = user.astype(jnp.int32)
    sess, xpn, win = _sc_gather_winner(
        user_sessions, W_emb, user, input_item.astype(jnp.int32),
        pos_items.astype(jnp.int32), neg_items.astype(jnp.int32))
    new_rows, scores = _tc_gru(sess, xpn, w_ih, w_hh, b_ih, b_hh)
    upd = _tc_copy(user_sessions)
    return scores, _sc_scatter(upd, user, win, new_rows)


# speedup vs baseline: 1.1062x; 1.0763x over previous
import functools

import jax
import jax.numpy as jnp
from jax import lax
from jax.experimental import pallas as pl
from jax.experimental.pallas import tpu as pltpu
from jax.experimental.pallas import tpu_sc as plsc

F32 = jnp.float32
_NW = 32
_CH = 128
_RNG = 3200


def _mesh():
    return plsc.VectorSubcoreMesh(core_axis_name="c", subcore_axis_name="s")


def _wid():
    return lax.axis_index("s") * 2 + lax.axis_index("c")


def _sc_gather_winner(sess_tab, W_emb, user, input_item, pos_items, neg_items):
    B = user.shape[0]
    bw_u = B // _NW
    bw_i = B // _NW
    n_chunks = (bw_u + 3 * bw_i) // _CH
    n_vec = B // 16
    seg = n_vec // n_chunks

    @functools.partial(
        pl.kernel,
        out_type=(
            jax.ShapeDtypeStruct((B, 2, 128), F32),
            jax.ShapeDtypeStruct((3 * B, 128), F32),
            jax.ShapeDtypeStruct((_NW * _RNG,), jnp.int32),
        ),
        mesh=_mesh(),
        compiler_params=pltpu.CompilerParams(needs_layout_passes=False),
        scratch_types=[
            pltpu.VMEM((bw_u,), jnp.int32),
            pltpu.VMEM((3 * bw_i,), jnp.int32),
            pltpu.VMEM((B,), jnp.int32),
            pltpu.VMEM((_RNG,), jnp.int32),
            pltpu.VMEM((2, _CH, 2, 128), F32),
            pltpu.VMEM((2, _CH, 128), F32),
            pltpu.SemaphoreType.DMA,
            pltpu.SemaphoreType.DMA,
            pltpu.SemaphoreType.DMA,
            pltpu.SemaphoreType.DMA,
        ],
    )
    def k(sess_hbm, emb_hbm, user_hbm, in_hbm, pos_hbm, neg_hbm,
          sess_out, xpn_out, win_out,
          uidx, iidx, uall, wloc, sbuf, ebuf, g0, g1, w0, w1):
        wid = _wid()
        ubase = pl.multiple_of(wid * bw_u, _CH)
        ibase = pl.multiple_of(wid * bw_i, _CH)
        lo = pl.multiple_of(wid * _RNG, _RNG)
        pltpu.sync_copy(user_hbm.at[pl.ds(ubase, bw_u)], uidx)
        for r, hbm in enumerate((in_hbm, pos_hbm, neg_hbm)):
            pltpu.sync_copy(hbm.at[pl.ds(ibase, bw_i)],
                            iidx.at[pl.ds(r * bw_i, bw_i)])
        pltpu.sync_copy(user_hbm, uall)
        gsem = (g0, g1)
        wsem = (w0, w1)

        lane = lax.iota(jnp.int32, 16)
        perm = lax.bitwise_and(lane + 1, 15)

        def win_body(j, carry):
            u = uall[pl.ds(pl.multiple_of(j * 16, 16), 16)]
            ps = j * 16 + lane
            inr = jnp.logical_and(u >= lo, u < lo + _RNG)
            plsc.store_scatter(wloc, [u - lo], ps, mask=inr)
            return carry

        seg_no = [0]

        def winner_segment():
            t = seg_no[0]
            if t < n_chunks:
                @pl.loop(t * seg, (t + 1) * seg, unroll=2)
                def _(j):
                    win_body(j, 0)
            seg_no[0] += 1

        def pipeline(n, tab, idx, buf, out, obase_fn):
            gh = [None] * n
            wh = [None] * n
            gh[0] = pltpu.async_copy(tab.at[idx.at[pl.ds(0, _CH)]], buf.at[0], gsem[0])
            for j in range(n):
                b = j & 1
                gh[j].wait()
                dst = out.at[pl.ds(pl.multiple_of(obase_fn(j), _CH), _CH)]
                wh[j] = pltpu.async_copy(buf.at[b], dst, wsem[b])
                if j + 1 < n:
                    if j >= 1:
                        wh[j - 1].wait()
                    gh[j + 1] = pltpu.async_copy(
                        tab.at[idx.at[pl.ds((j + 1) * _CH, _CH)]],
                        buf.at[(j + 1) & 1], gsem[(j + 1) & 1])
                winner_segment()
            if n >= 2:
                wh[n - 2].wait()
            wh[n - 1].wait()

        nu = bw_u // _CH
        ni = bw_i // _CH
        pipeline(nu, sess_hbm, uidx, sbuf, sess_out,
                 lambda j: ubase + j * _CH)
        pipeline(3 * ni, emb_hbm, iidx, ebuf, xpn_out,
                 lambda j: (j // ni) * B + ibase + (j % ni) * _CH)
        pltpu.sync_copy(wloc, win_out.at[pl.ds(lo, _RNG)])

    return k(sess_tab, W_emb, user, input_item, pos_items, neg_items)


def _tc_gru(sess, xpn, w_ih, w_hh, b_ih, b_hh):
    B = sess.shape[0]
    BB = 2048
    nb = B // BB
    dn = (((1,), (1,)), ((), ()))

    def body(sess_ref, x_ref, p_ref, n_ref, wih_ref, whh_ref, bih_ref, bhh_ref,
             out_ref, sc_ref):
        h_in = x_ref[...]
        hs = []
        for l in range(2):
            h_prev = sess_ref[:, l, :]
            gi = lax.dot_general(h_in, wih_ref[l], dn,
                                 preferred_element_type=F32) + bih_ref[l][None, :]
            gh = lax.dot_general(h_prev, whh_ref[l], dn,
                                 preferred_element_type=F32) + bhh_ref[l][None, :]
            r = jax.nn.sigmoid(gi[:, 0:128] + gh[:, 0:128])
            z = jax.nn.sigmoid(gi[:, 128:256] + gh[:, 128:256])
            n = jnp.tanh(gi[:, 256:384] + r * gh[:, 256:384])
            h_in = (1.0 - z) * n + z * h_prev
            hs.append(h_in)
        out_ref[:, 0, :] = hs[0]
        out_ref[:, 1, :] = hs[1]
        sc_ref[...] = jnp.sum(h_in * (p_ref[...] - n_ref[...]),
                              axis=-1, keepdims=True)

    return pl.pallas_call(
        body,
        grid=(nb,),
        in_specs=[
            pl.BlockSpec((BB, 2, 128), lambda i: (i, 0, 0)),
            pl.BlockSpec((BB, 128), lambda i: (i, 0)),
            pl.BlockSpec((BB, 128), lambda i, _nb=nb: (i + _nb, 0)),
            pl.BlockSpec((BB, 128), lambda i, _nb=nb: (i + 2 * _nb, 0)),
            pl.BlockSpec((2, 384, 128), lambda i: (0, 0, 0)),
            pl.BlockSpec((2, 384, 128), lambda i: (0, 0, 0)),
            pl.BlockSpec((2, 384), lambda i: (0, 0)),
            pl.BlockSpec((2, 384), lambda i: (0, 0)),
        ],
        out_specs=[
            pl.BlockSpec((BB, 2, 128), lambda i: (i, 0, 0)),
            pl.BlockSpec((BB, 1), lambda i: (i, 0)),
        ],
        out_shape=[
            jax.ShapeDtypeStruct((B, 2, 128), F32),
            jax.ShapeDtypeStruct((B, 1), F32),
        ],
    )(sess, xpn, xpn, xpn, w_ih, w_hh, b_ih, b_hh)


def _tc_copy(tab):
    R = tab.shape[0]
    BR = 5000

    def body(in_ref, out_ref):
        out_ref[...] = in_ref[...]

    return pl.pallas_call(
        body,
        grid=(R // BR,),
        in_specs=[pl.BlockSpec((BR, 2, 128), lambda i: (i, 0, 0))],
        out_specs=pl.BlockSpec((BR, 2, 128), lambda i: (i, 0, 0)),
        out_shape=jax.ShapeDtypeStruct((R, 2, 128), F32),
    )(tab)


def _sc_scatter(upd, user, win, new_rows):
    B = user.shape[0]
    bw = B // _NW

    n = bw // _CH

    @functools.partial(
        pl.kernel,
        mesh=_mesh(),
        scratch_types=[
            pltpu.VMEM((n, _CH), jnp.int32),
            pltpu.VMEM((n, _CH), jnp.int32),
            pltpu.VMEM((2, _CH, 2, 128), F32),
            pltpu.SemaphoreType.DMA,
            pltpu.SemaphoreType.DMA,
        ],
    )
    def k(out_hbm, user_hbm, win_hbm, rows_hbm, uidx, sel, row, sem0, sem1):
        wid = _wid()
        sems = (sem0, sem1)

        for j in range(n):
            base = pl.multiple_of(wid * bw + j * _CH, _CH)
            pltpu.sync_copy(user_hbm.at[pl.ds(base, _CH)], uidx.at[j])
        sh = [pltpu.async_copy(win_hbm.at[uidx.at[j]], sel.at[j], sem0)
              for j in range(n)]
        for h in sh:
            h.wait()

        gh = [None] * n
        wh = [None] * n
        gh[0] = pltpu.async_copy(rows_hbm.at[sel.at[0]], row.at[0], sems[0])
        for j in range(n):
            b = j & 1
            gh[j].wait()
            wh[j] = pltpu.async_copy(row.at[b], out_hbm.at[uidx.at[j]], sems[b])
            if j + 1 < n:
                if j >= 1:
                    wh[j - 1].wait()
                gh[j + 1] = pltpu.async_copy(
                    rows_hbm.at[sel.at[j + 1]], row.at[(j + 1) & 1],
                    sems[(j + 1) & 1])
        if n >= 2:
            wh[n - 2].wait()
        wh[n - 1].wait()

    ref = jax.new_ref(upd)
    k(ref, user, win, new_rows)
    return ref[...]


def kernel(user, input_item, pos_items, neg_items, user_sessions, W_emb,
           w_ih, w_hh, b_ih, b_hh):
    user = user.astype(jnp.int32)
    sess, xpn, win = _sc_gather_winner(
        user_sessions, W_emb, user, input_item.astype(jnp.int32),
        pos_items.astype(jnp.int32), neg_items.astype(jnp.int32))
    new_rows, scores = _tc_gru(sess, xpn, w_ih, w_hh, b_ih, b_hh)
    upd = _tc_copy(user_sessions)
    return scores, _sc_scatter(upd, user, win, new_rows)


# speedup vs baseline: 1.1135x; 1.0066x over previous
import functools

import jax
import jax.numpy as jnp
from jax import lax
from jax.experimental import pallas as pl
from jax.experimental.pallas import tpu as pltpu
from jax.experimental.pallas import tpu_sc as plsc

F32 = jnp.float32
_NW = 32
_CH = 128
_RNG = 3200


def _mesh():
    return plsc.VectorSubcoreMesh(core_axis_name="c", subcore_axis_name="s")


def _wid():
    return lax.axis_index("s") * 2 + lax.axis_index("c")


def _sc_gather_winner(sess_tab, W_emb, user, input_item, pos_items, neg_items):
    B = user.shape[0]
    bw_u = B // _NW
    bw_i = B // _NW
    n_chunks = (bw_u + 3 * bw_i) // _CH
    n_vec = B // 16
    seg = n_vec // n_chunks

    @functools.partial(
        pl.kernel,
        out_type=(
            jax.ShapeDtypeStruct((B, 2, 128), F32),
            jax.ShapeDtypeStruct((3 * B, 128), F32),
            jax.ShapeDtypeStruct((_NW * _RNG,), jnp.int32),
        ),
        mesh=_mesh(),
        compiler_params=pltpu.CompilerParams(needs_layout_passes=False),
        scratch_types=[
            pltpu.VMEM((bw_u,), jnp.int32),
            pltpu.VMEM((3 * bw_i,), jnp.int32),
            pltpu.VMEM((B,), jnp.int32),
            pltpu.VMEM((_RNG,), jnp.int32),
            pltpu.VMEM((2, _CH, 2, 128), F32),
            pltpu.VMEM((2, _CH, 128), F32),
            pltpu.SemaphoreType.DMA,
            pltpu.SemaphoreType.DMA,
            pltpu.SemaphoreType.DMA,
            pltpu.SemaphoreType.DMA,
        ],
    )
    def k(sess_hbm, emb_hbm, user_hbm, in_hbm, pos_hbm, neg_hbm,
          sess_out, xpn_out, win_out,
          uidx, iidx, uall, wloc, sbuf, ebuf, g0, g1, w0, w1):
        wid = _wid()
        ubase = pl.multiple_of(wid * bw_u, _CH)
        ibase = pl.multiple_of(wid * bw_i, _CH)
        lo = pl.multiple_of(wid * _RNG, _RNG)
        pltpu.sync_copy(user_hbm.at[pl.ds(ubase, bw_u)], uidx)
        for r, hbm in enumerate((in_hbm, pos_hbm, neg_hbm)):
            pltpu.sync_copy(hbm.at[pl.ds(ibase, bw_i)],
                            iidx.at[pl.ds(r * bw_i, bw_i)])
        pltpu.sync_copy(user_hbm, uall)
        gsem = (g0, g1)
        wsem = (w0, w1)

        lane = lax.iota(jnp.int32, 16)
        perm = lax.bitwise_and(lane + 1, 15)

        def win_body(j, carry):
            u = uall[pl.ds(pl.multiple_of(j * 16, 16), 16)]
            ps = j * 16 + lane
            inr = jnp.logical_and(u >= lo, u < lo + _RNG)
            plsc.store_scatter(wloc, [u - lo], ps, mask=inr)
            return carry

        seg_no = [0]

        def winner_segment():
            t = seg_no[0]
            if t < n_chunks:
                @pl.loop(t * seg, (t + 1) * seg, unroll=4)
                def _(j):
                    win_body(j, 0)
            seg_no[0] += 1

        def pipeline(n, tab, idx, buf, out, obase_fn):
            gh = [None] * n
            wh = [None] * n
            gh[0] = pltpu.async_copy(tab.at[idx.at[pl.ds(0, _CH)]], buf.at[0], gsem[0])
            for j in range(n):
                b = j & 1
                gh[j].wait()
                dst = out.at[pl.ds(pl.multiple_of(obase_fn(j), _CH), _CH)]
                wh[j] = pltpu.async_copy(buf.at[b], dst, wsem[b])
                if j + 1 < n:
                    if j >= 1:
                        wh[j - 1].wait()
                    gh[j + 1] = pltpu.async_copy(
                        tab.at[idx.at[pl.ds((j + 1) * _CH, _CH)]],
                        buf.at[(j + 1) & 1], gsem[(j + 1) & 1])
                winner_segment()
            if n >= 2:
                wh[n - 2].wait()
            wh[n - 1].wait()

        nu = bw_u // _CH
        ni = bw_i // _CH
        pipeline(nu, sess_hbm, uidx, sbuf, sess_out,
                 lambda j: ubase + j * _CH)
        pipeline(3 * ni, emb_hbm, iidx, ebuf, xpn_out,
                 lambda j: (j // ni) * B + ibase + (j % ni) * _CH)
        pltpu.sync_copy(wloc, win_out.at[pl.ds(lo, _RNG)])

    return k(sess_tab, W_emb, user, input_item, pos_items, neg_items)


def _tc_gru(sess, xpn, w_ih, w_hh, b_ih, b_hh):
    B = sess.shape[0]
    BB = 2048
    nb = B // BB
    dn = (((1,), (1,)), ((), ()))

    def body(sess_ref, x_ref, p_ref, n_ref, wih_ref, whh_ref, bih_ref, bhh_ref,
             out_ref, sc_ref):
        h_in = x_ref[...]
        hs = []
        for l in range(2):
            h_prev = sess_ref[:, l, :]
            gi = lax.dot_general(h_in, wih_ref[l], dn,
                                 preferred_element_type=F32) + bih_ref[l][None, :]
            gh = lax.dot_general(h_prev, whh_ref[l], dn,
                                 preferred_element_type=F32) + bhh_ref[l][None, :]
            r = jax.nn.sigmoid(gi[:, 0:128] + gh[:, 0:128])
            z = jax.nn.sigmoid(gi[:, 128:256] + gh[:, 128:256])
            n = jnp.tanh(gi[:, 256:384] + r * gh[:, 256:384])
            h_in = (1.0 - z) * n + z * h_prev
            hs.append(h_in)
        out_ref[:, 0, :] = hs[0]
        out_ref[:, 1, :] = hs[1]
        sc_ref[...] = jnp.sum(h_in * (p_ref[...] - n_ref[...]),
                              axis=-1, keepdims=True)

    return pl.pallas_call(
        body,
        grid=(nb,),
        in_specs=[
            pl.BlockSpec((BB, 2, 128), lambda i: (i, 0, 0)),
            pl.BlockSpec((BB, 128), lambda i: (i, 0)),
            pl.BlockSpec((BB, 128), lambda i, _nb=nb: (i + _nb, 0)),
            pl.BlockSpec((BB, 128), lambda i, _nb=nb: (i + 2 * _nb, 0)),
            pl.BlockSpec((2, 384, 128), lambda i: (0, 0, 0)),
            pl.BlockSpec((2, 384, 128), lambda i: (0, 0, 0)),
            pl.BlockSpec((2, 384), lambda i: (0, 0)),
            pl.BlockSpec((2, 384), lambda i: (0, 0)),
        ],
        out_specs=[
            pl.BlockSpec((BB, 2, 128), lambda i: (i, 0, 0)),
            pl.BlockSpec((BB, 1), lambda i: (i, 0)),
        ],
        out_shape=[
            jax.ShapeDtypeStruct((B, 2, 128), F32),
            jax.ShapeDtypeStruct((B, 1), F32),
        ],
    )(sess, xpn, xpn, xpn, w_ih, w_hh, b_ih, b_hh)


def _tc_copy(tab):
    R = tab.shape[0]
    BR = 5000

    def body(in_ref, out_ref):
        out_ref[...] = in_ref[...]

    return pl.pallas_call(
        body,
        grid=(R // BR,),
        in_specs=[pl.BlockSpec((BR, 2, 128), lambda i: (i, 0, 0))],
        out_specs=pl.BlockSpec((BR, 2, 128), lambda i: (i, 0, 0)),
        out_shape=jax.ShapeDtypeStruct((R, 2, 128), F32),
    )(tab)


def _sc_scatter(upd, user, win, new_rows):
    B = user.shape[0]
    bw = B // _NW

    n = bw // _CH

    @functools.partial(
        pl.kernel,
        mesh=_mesh(),
        scratch_types=[
            pltpu.VMEM((n, _CH), jnp.int32),
            pltpu.VMEM((n, _CH), jnp.int32),
            pltpu.VMEM((2, _CH, 2, 128), F32),
            pltpu.SemaphoreType.DMA,
            pltpu.SemaphoreType.DMA,
        ],
    )
    def k(out_hbm, user_hbm, win_hbm, rows_hbm, uidx, sel, row, sem0, sem1):
        wid = _wid()
        sems = (sem0, sem1)

        uh = [pltpu.async_copy(
                  user_hbm.at[pl.ds(pl.multiple_of(wid * bw + j * _CH, _CH), _CH)],
                  uidx.at[j], sem1)
              for j in range(n)]
        for h in uh:
            h.wait()
        sh = [pltpu.async_copy(win_hbm.at[uidx.at[j]], sel.at[j], sem0)
              for j in range(n)]
        for h in sh:
            h.wait()

        gh = [None] * n
        wh = [None] * n
        gh[0] = pltpu.async_copy(rows_hbm.at[sel.at[0]], row.at[0], sems[0])
        for j in range(n):
            b = j & 1
            gh[j].wait()
            wh[j] = pltpu.async_copy(row.at[b], out_hbm.at[uidx.at[j]], sems[b])
            if j + 1 < n:
                if j >= 1:
                    wh[j - 1].wait()
                gh[j + 1] = pltpu.async_copy(
                    rows_hbm.at[sel.at[j + 1]], row.at[(j + 1) & 1],
                    sems[(j + 1) & 1])
        if n >= 2:
            wh[n - 2].wait()
        wh[n - 1].wait()

    ref = jax.new_ref(upd)
    k(ref, user, win, new_rows)
    return ref[...]


def kernel(user, input_item, pos_items, neg_items, user_sessions, W_emb,
           w_ih, w_hh, b_ih, b_hh):
    user = user.astype(jnp.int32)
    sess, xpn, win = _sc_gather_winner(
        user_sessions, W_emb, user, input_item.astype(jnp.int32),
        pos_items.astype(jnp.int32), neg_items.astype(jnp.int32))
    new_rows, scores = _tc_gru(sess, xpn, w_ih, w_hh, b_ih, b_hh)
    upd = _tc_copy(user_sessions)
    return scores, _sc_scatter(upd, user, win, new_rows)
